# Initial kernel scaffold; baseline (speedup 1.0000x reference)
import jax
import jax.numpy as jnp
from jax.experimental import pallas as pl


def kernel(FF, V, b_adapt, X_prev, w_int, w_ff, edge_index_int, edge_index_ff):
    raise NotImplementedError("write your pallas kernel here")


# speedup vs baseline: 133.3237x; 133.3237x over previous
import functools

import jax
import jax.numpy as jnp
from jax import lax
from jax.experimental import pallas as pl
from jax.experimental.pallas import tpu as pltpu
from jax.experimental.pallas import tpu_sc as plsc

N = 50000
E = 1_600_000
NPAD = 50176
NROWS2D = 392
THETA0 = 1.0
BETA = 1.8
ALPHA = 0.9
RHO = 0.99
SLOPE = 10.0

NC = 2
NS = 16
L = 16

ROW = 128
ROWS = 16
CHUNK = ROWS * ROW
NCHUNKS = (E + CHUNK - 1) // CHUNK
TAIL_K = NCHUNKS - 1
TAIL_ROWS = (E - TAIL_K * CHUNK) // ROW
TPC = (NCHUNKS + NS - 1) // NS

ZSLICE = 2000
NZ = N // ZSLICE


def _emit_row(r, src_buf, dst_buf, w_buf, table, dst2d, contrib2d):
    for i in range(ROW // L):
        sl = pl.ds(r * ROW + i * L, L)
        sv = src_buf[sl]
        dv = dst_buf[sl]
        wv = w_buf[sl]
        xv = plsc.load_gather(table, [sv])
        contrib2d[r, pl.ds(i * L, L)] = wv * xv
        dst2d[r, pl.ds(i * L, L)] = dv


def _zero_row(r, contrib2d):
    z = jnp.zeros((L,), jnp.float32)
    for i in range(ROW // L):
        contrib2d[r, pl.ds(i * L, L)] = z


def _edge_pass(src_hbm, dst_hbm, w_hbm, table, shared, src_buf, dst_buf,
               w_buf, dst2d, contrib2d, sem, sid):
    def chunk_body(t, carry):
        k = sid + t * NS

        @pl.when(k < NCHUNKS)
        def _():
            off = k * CHUNK
            is_tail = k == TAIL_K

            @pl.when(jnp.logical_not(is_tail))
            def _():
                pltpu.sync_copy(src_hbm.at[pl.ds(off, CHUNK)],
                                src_buf.at[pl.ds(0, CHUNK)])
                pltpu.sync_copy(dst_hbm.at[pl.ds(off, CHUNK)],
                                dst_buf.at[pl.ds(0, CHUNK)])
                pltpu.sync_copy(w_hbm.at[pl.ds(off, CHUNK)],
                                w_buf.at[pl.ds(0, CHUNK)])

            @pl.when(is_tail)
            def _():
                tn = TAIL_ROWS * ROW
                pltpu.sync_copy(src_hbm.at[pl.ds(off, tn)],
                                src_buf.at[pl.ds(0, tn)])
                pltpu.sync_copy(dst_hbm.at[pl.ds(off, tn)],
                                dst_buf.at[pl.ds(0, tn)])
                pltpu.sync_copy(w_hbm.at[pl.ds(off, tn)],
                                w_buf.at[pl.ds(0, tn)])

            for r in range(ROWS):
                if r < TAIL_ROWS:
                    _emit_row(r, src_buf, dst_buf, w_buf, table, dst2d,
                              contrib2d)
                else:
                    @pl.when(jnp.logical_not(is_tail))
                    def _(r=r):
                        _emit_row(r, src_buf, dst_buf, w_buf, table, dst2d,
                                  contrib2d)

                    @pl.when(is_tail)
                    def _(r=r):
                        _zero_row(r, contrib2d)

            descs = []
            for r in range(ROWS):
                descs.append(
                    pltpu.async_copy(contrib2d.at[r],
                                     shared.at[dst2d.at[r]],
                                     sem, add=True))
            for d in descs:
                d.wait()

        return carry

    lax.fori_loop(0, TPC, chunk_body, 0)


def _sc_body(edges_i, w_i, edges_f, w_f, xprev, ffin,
             out_i, out_f,
             table, src_buf, dst_buf, w_buf, dst2d, contrib2d, zbuf,
             shared, sem):
    cid = lax.axis_index("c")
    sid = lax.axis_index("s")

    z = jnp.zeros((L,), jnp.float32)
    for i in range(ZSLICE // L):
        zbuf[pl.ds(i * L, L)] = z
    pltpu.sync_copy(zbuf, shared.at[pl.ds(sid * ZSLICE, ZSLICE)])

    @pl.when(sid < NZ - NS)
    def _():
        pltpu.sync_copy(zbuf, shared.at[pl.ds((sid + NS) * ZSLICE,
                                              ZSLICE)])

    @pl.when(cid == 0)
    def _():
        pltpu.sync_copy(xprev, table)

    @pl.when(cid == 1)
    def _():
        pltpu.sync_copy(ffin, table)

    plsc.subcore_barrier()

    @pl.when(cid == 0)
    def _():
        _edge_pass(edges_i.at[0], edges_i.at[1], w_i, table, shared,
                   src_buf, dst_buf, w_buf, dst2d, contrib2d, sem, sid)

    @pl.when(cid == 1)
    def _():
        _edge_pass(edges_f.at[0], edges_f.at[1], w_f, table, shared,
                   src_buf, dst_buf, w_buf, dst2d, contrib2d, sem, sid)

    plsc.subcore_barrier()

    def write_out(out):
        pltpu.sync_copy(shared.at[pl.ds(sid * ZSLICE, ZSLICE)], zbuf)
        pltpu.sync_copy(zbuf, out.at[pl.ds(sid * ZSLICE, ZSLICE)])

        @pl.when(sid < NZ - NS)
        def _():
            pltpu.sync_copy(shared.at[pl.ds((sid + NS) * ZSLICE, ZSLICE)],
                            zbuf)
            pltpu.sync_copy(zbuf, out.at[pl.ds((sid + NS) * ZSLICE, ZSLICE)])

    @pl.when(cid == 0)
    def _():
        write_out(out_i)

    @pl.when(cid == 1)
    def _():
        write_out(out_f)


_sc_currents = pl.kernel(
    _sc_body,
    out_type=(
        jax.ShapeDtypeStruct((NPAD,), jnp.float32),
        jax.ShapeDtypeStruct((NPAD,), jnp.float32),
    ),
    mesh=plsc.VectorSubcoreMesh(core_axis_name="c", subcore_axis_name="s",
                                num_cores=NC, num_subcores=NS),
    scratch_types=[
        pltpu.VMEM((N,), jnp.float32),
        pltpu.VMEM((CHUNK,), jnp.int32),
        pltpu.VMEM((CHUNK,), jnp.int32),
        pltpu.VMEM((CHUNK,), jnp.float32),
        pltpu.VMEM((ROWS, ROW), jnp.int32),
        pltpu.VMEM((ROWS, ROW), jnp.float32),
        pltpu.VMEM((ZSLICE,), jnp.float32),
        pltpu.VMEM_SHARED((N,), jnp.float32),
        pltpu.SemaphoreType.DMA,
    ],
    compiler_params=pltpu.CompilerParams(needs_layout_passes=False),
)


def _tc_body(v_ref, b_ref, ci_ref, cf_ref, x_ref, vn_ref, bn_ref):
    v = v_ref[...]
    b = b_ref[...]
    thr = THETA0 + BETA * b
    x = jax.nn.sigmoid(SLOPE * (v - thr))
    cur = ci_ref[...] + cf_ref[...]
    x_ref[...] = x
    vn_ref[...] = ALPHA * v * (1.0 - x) + cur
    bn_ref[...] = RHO * b + (1.0 - RHO) * x


_tc_update = pl.pallas_call(
    _tc_body,
    out_shape=[
        jax.ShapeDtypeStruct((NROWS2D, 128), jnp.float32),
        jax.ShapeDtypeStruct((NROWS2D, 128), jnp.float32),
        jax.ShapeDtypeStruct((NROWS2D, 128), jnp.float32),
    ],
)


def kernel(FF, V, b_adapt, X_prev, w_int, w_ff, edge_index_int,
           edge_index_ff):
    cur_i, cur_f = _sc_currents(edge_index_int, w_int, edge_index_ff, w_ff,
                                X_prev, FF)

    pad = NPAD - N
    v2 = jnp.pad(V, (0, pad)).reshape(NROWS2D, 128)
    b2 = jnp.pad(b_adapt, (0, pad)).reshape(NROWS2D, 128)
    x2, vn2, bn2 = _tc_update(v2, b2, cur_i.reshape(NROWS2D, 128),
                              cur_f.reshape(NROWS2D, 128))
    X = x2.reshape(-1)[:N]
    V_new = vn2.reshape(-1)[:N]
    b_new = bn2.reshape(-1)[:N]
    return (X, V_new, b_new)


# speedup vs baseline: 135.8605x; 1.0190x over previous
import jax
import jax.numpy as jnp
from jax import lax
from jax.experimental import pallas as pl
from jax.experimental.pallas import tpu as pltpu
from jax.experimental.pallas import tpu_sc as plsc

N = 50000
E = 1_600_000
NPAD = 50176
NROWS2D = 392
THETA0 = 1.0
BETA = 1.8
ALPHA = 0.9
RHO = 0.99
SLOPE = 10.0

NC = 2
NS = 16
L = 16

CHUNK = 2048
NSTEP = CHUNK // L
U = 8
NCHUNKS = (E + CHUNK - 1) // CHUNK
TAIL_K = NCHUNKS - 1
TAIL_EDGES = E - TAIL_K * CHUNK
TAIL_J0 = (CHUNK - TAIL_EDGES) // (L * U)
TPC = (NCHUNKS + NS - 1) // NS
TPC2 = (TPC + 1) // 2


def _edge_pass(src_hbm, dst_hbm, w_hbm, table, acc, bufs, sems, sid):
    (src0, dst0, w0), (src1, dst1, w1) = bufs
    semA, semB = sems

    def issue(k, sb, db, wb, sem):
        @pl.when(k < NCHUNKS)
        def _():
            off = jnp.where(k == TAIL_K, E - CHUNK, k * CHUNK)
            pltpu.async_copy(src_hbm.at[pl.ds(off, CHUNK)], sb, sem)
            pltpu.async_copy(dst_hbm.at[pl.ds(off, CHUNK)], db, sem)
            pltpu.async_copy(w_hbm.at[pl.ds(off, CHUNK)], wb, sem)

    def wait3(k, sb, db, wb, sem):
        @pl.when(k < NCHUNKS)
        def _():
            pltpu.make_async_copy(src_hbm.at[pl.ds(0, CHUNK)], sb, sem).wait()
            pltpu.make_async_copy(dst_hbm.at[pl.ds(0, CHUNK)], db, sem).wait()
            pltpu.make_async_copy(w_hbm.at[pl.ds(0, CHUNK)], wb, sem).wait()

    def compute(k, sb, db, wb):
        @pl.when(k < NCHUNKS)
        def _():
            j0 = jnp.where(k == TAIL_K, TAIL_J0, 0)

            def step(jj, c):
                for u in range(U):
                    sl = pl.ds(jj * (U * L) + u * L, L)
                    sv = sb[sl]
                    dv = db[sl]
                    wv = wb[sl]
                    xv = plsc.load_gather(table, [sv])
                    plsc.addupdate_scatter(acc, [dv], wv * xv)
                return c

            lax.fori_loop(j0, NSTEP // U, step, 0)

    issue(sid, src0, dst0, w0, semA)

    def body(t, c):
        kA = sid + (2 * t) * NS
        kB = sid + (2 * t + 1) * NS
        wait3(kA, src0, dst0, w0, semA)
        issue(kB, src1, dst1, w1, semB)
        compute(kA, src0, dst0, w0)
        wait3(kB, src1, dst1, w1, semB)

        @pl.when(t + 1 < TPC2)
        def _():
            issue(sid + (2 * t + 2) * NS, src0, dst0, w0, semA)

        compute(kB, src1, dst1, w1)
        return c

    lax.fori_loop(0, TPC2, body, 0)


def _sc_body(edges_i, w_i, edges_f, w_f, xprev, ffin, *refs):
    outs = refs[:NC * NS]
    (table, acc, src0, dst0, w0, src1, dst1, w1, semA, semB) = refs[NC * NS:]
    cid = lax.axis_index("c")
    sid = lax.axis_index("s")

    @pl.when(cid == 0)
    def _():
        pltpu.sync_copy(xprev, table)

    @pl.when(cid == 1)
    def _():
        pltpu.sync_copy(ffin, table)

    z = jnp.zeros((L,), jnp.float32)

    def zbody(i, c):
        for u in range(5):
            acc[pl.ds((i * 5 + u) * L, L)] = z
        return c

    lax.fori_loop(0, N // (L * 5), zbody, 0)

    bufs = ((src0, dst0, w0), (src1, dst1, w1))
    sems = (semA, semB)

    @pl.when(cid == 0)
    def _():
        _edge_pass(edges_i.at[0], edges_i.at[1], w_i, table, acc, bufs,
                   sems, sid)

    @pl.when(cid == 1)
    def _():
        _edge_pass(edges_f.at[0], edges_f.at[1], w_f, table, acc, bufs,
                   sems, sid)

    for c in range(NC):
        for s in range(NS):
            @pl.when((cid == c) & (sid == s))
            def _(c=c, s=s):
                pltpu.sync_copy(acc, outs[c * NS + s].at[pl.ds(0, N)])


_sc_currents = pl.kernel(
    _sc_body,
    out_type=tuple(jax.ShapeDtypeStruct((NPAD,), jnp.float32)
                   for _ in range(NC * NS)),
    mesh=plsc.VectorSubcoreMesh(core_axis_name="c", subcore_axis_name="s",
                                num_cores=NC, num_subcores=NS),
    scratch_types=[
        pltpu.VMEM((N,), jnp.float32),
        pltpu.VMEM((N,), jnp.float32),
        pltpu.VMEM((CHUNK,), jnp.int32),
        pltpu.VMEM((CHUNK,), jnp.int32),
        pltpu.VMEM((CHUNK,), jnp.float32),
        pltpu.VMEM((CHUNK,), jnp.int32),
        pltpu.VMEM((CHUNK,), jnp.int32),
        pltpu.VMEM((CHUNK,), jnp.float32),
        pltpu.SemaphoreType.DMA,
        pltpu.SemaphoreType.DMA,
    ],
    compiler_params=pltpu.CompilerParams(needs_layout_passes=False,
                                         use_tc_tiling_on_sc=False),
)


def _tc_body(v_ref, b_ref, parts_ref, x_ref, vn_ref, bn_ref):
    v = v_ref[...]
    b = b_ref[...]
    thr = THETA0 + BETA * b
    x = jax.nn.sigmoid(SLOPE * (v - thr))
    cur = jnp.sum(parts_ref[...], axis=0)
    x_ref[...] = x
    vn_ref[...] = ALPHA * v * (1.0 - x) + cur
    bn_ref[...] = RHO * b + (1.0 - RHO) * x


_tc_update = pl.pallas_call(
    _tc_body,
    out_shape=[
        jax.ShapeDtypeStruct((NROWS2D, 128), jnp.float32),
        jax.ShapeDtypeStruct((NROWS2D, 128), jnp.float32),
        jax.ShapeDtypeStruct((NROWS2D, 128), jnp.float32),
    ],
)


def kernel(FF, V, b_adapt, X_prev, w_int, w_ff, edge_index_int,
           edge_index_ff):
    parts_list = _sc_currents(edge_index_int, w_int, edge_index_ff,
                              w_ff, X_prev, FF)
    parts = jnp.stack(parts_list).reshape(NC * NS, NROWS2D, 128)

    pad = NPAD - N
    v2 = jnp.pad(V, (0, pad)).reshape(NROWS2D, 128)
    b2 = jnp.pad(b_adapt, (0, pad)).reshape(NROWS2D, 128)
    x2, vn2, bn2 = _tc_update(v2, b2, parts)
    X = x2.reshape(-1)[:N]
    V_new = vn2.reshape(-1)[:N]
    b_new = bn2.reshape(-1)[:N]
    return (X, V_new, b_new)
